# Initial kernel scaffold; baseline (speedup 1.0000x reference)
import functools

import jax
import jax.numpy as jnp
from jax import lax
from jax.experimental import pallas as pl
from jax.experimental.pallas import tpu as pltpu
from jax.experimental.pallas import tpu_sc as plsc

VOCAB = 1000
HIDDEN = 128
EPS = 1e-12

_NC = 2
_NS = 16
_NW = _NC * _NS

_N_TOKENS = 4096 * 200
_PER_W = _N_TOKENS // _NW
_GBATCH = 128
_CHUNK = 512
_NGB = _CHUNK // _GBATCH
_N_CHUNKS = _PER_W // _CHUNK


def _ln_body(table_ref, gamma_ref, beta_ref, out_ref):
    x = table_ref[...]
    mean = jnp.mean(x, axis=1, keepdims=True)
    cent = x - mean
    var = jnp.mean(cent * cent, axis=1, keepdims=True)
    normed = cent * lax.rsqrt(var + EPS)
    out_ref[...] = normed * gamma_ref[...] + beta_ref[...]


def _normalize_table(table, gamma, beta):
    return pl.pallas_call(
        _ln_body,
        out_shape=jax.ShapeDtypeStruct((VOCAB, HIDDEN), jnp.float32),
    )(table, gamma.reshape(1, HIDDEN), beta.reshape(1, HIDDEN))


@functools.partial(
    pl.kernel,
    mesh=plsc.VectorSubcoreMesh(core_axis_name="c", subcore_axis_name="s"),
    out_type=jax.ShapeDtypeStruct((_N_TOKENS, HIDDEN), jnp.float32),
    scratch_types=[
        pltpu.VMEM((_NGB, _GBATCH), jnp.int32),
        pltpu.VMEM((_CHUNK, HIDDEN), jnp.float32),
        pltpu.SemaphoreType.DMA,
    ],
)
def _gather_kernel(normed_hbm, idx_hbm, out_hbm, idx_v, rows_v, sem):
    wid = lax.axis_index("s") * _NC + lax.axis_index("c")
    base = wid * _PER_W

    def body(i, carry):
        off = base + i * _CHUNK
        pltpu.sync_copy(idx_hbm.at[pl.ds(off, _CHUNK)], idx_v)
        for j in range(_NGB):
            pltpu.async_copy(
                normed_hbm.at[idx_v.at[j]],
                rows_v.at[pl.ds(j * _GBATCH, _GBATCH)],
                sem,
            )
        for j in range(_NGB):
            pltpu.make_async_copy(
                normed_hbm.at[idx_v.at[j]],
                rows_v.at[pl.ds(j * _GBATCH, _GBATCH)],
                sem,
            ).wait()
        pltpu.sync_copy(rows_v, out_hbm.at[pl.ds(off, _CHUNK)])
        return carry

    lax.fori_loop(0, _N_CHUNKS, body, 0)


def kernel(input_ids, table, gamma, beta):
    normed = _normalize_table(table, gamma, beta)
    flat_ids = input_ids.reshape(-1).astype(jnp.int32)
    out = _gather_kernel(normed, flat_ids)
    return out.reshape(input_ids.shape + (HIDDEN,))


# speedup vs baseline: 8.4727x; 8.4727x over previous
import functools

import jax
import jax.numpy as jnp
from jax import lax
from jax.experimental import pallas as pl
from jax.experimental.pallas import tpu as pltpu
from jax.experimental.pallas import tpu_sc as plsc

VOCAB = 1000
HIDDEN = 128
EPS = 1e-12

_NC = 2
_NS = 16
_NW = _NC * _NS

_N_TOKENS = 4096 * 200
_PER_W = _N_TOKENS // _NW
_GBATCH = 128
_CHUNK = 512
_NGB = _CHUNK // _GBATCH
_N_CHUNKS = _PER_W // _CHUNK


def _ln_body(table_ref, gamma_ref, beta_ref, out_ref):
    x = table_ref[...]
    mean = jnp.mean(x, axis=1, keepdims=True)
    cent = x - mean
    var = jnp.mean(cent * cent, axis=1, keepdims=True)
    normed = cent * lax.rsqrt(var + EPS)
    out_ref[...] = normed * gamma_ref[...] + beta_ref[...]


def _normalize_table(table, gamma, beta):
    return pl.pallas_call(
        _ln_body,
        out_shape=jax.ShapeDtypeStruct((VOCAB, HIDDEN), jnp.float32),
    )(table, gamma.reshape(1, HIDDEN), beta.reshape(1, HIDDEN))


@functools.partial(
    pl.kernel,
    mesh=plsc.VectorSubcoreMesh(core_axis_name="c", subcore_axis_name="s"),
    out_type=jax.ShapeDtypeStruct((_N_TOKENS, HIDDEN), jnp.float32),
    scratch_types=[
        pltpu.VMEM((_CHUNK,), jnp.int32),
        pltpu.VMEM((_CHUNK, HIDDEN), jnp.float32),
        pltpu.SemaphoreType.DMA,
    ],
)
def _gather_kernel(normed_hbm, idx_hbm, out_hbm, idx_v, rows_v, sem):
    wid = lax.axis_index("s") * _NC + lax.axis_index("c")
    base = wid * _PER_W

    def body(i, carry):
        off = base + i * _CHUNK
        pltpu.sync_copy(idx_hbm.at[pl.ds(off, _CHUNK)], idx_v)
        for j in range(_NGB):
            pltpu.async_copy(
                normed_hbm.at[idx_v.at[pl.ds(j * _GBATCH, _GBATCH)]],
                rows_v.at[pl.ds(j * _GBATCH, _GBATCH)],
                sem,
            )
        for j in range(_NGB):
            pltpu.make_async_copy(
                normed_hbm.at[idx_v.at[pl.ds(j * _GBATCH, _GBATCH)]],
                rows_v.at[pl.ds(j * _GBATCH, _GBATCH)],
                sem,
            ).wait()
        pltpu.sync_copy(rows_v, out_hbm.at[pl.ds(off, _CHUNK)])
        return carry

    lax.fori_loop(0, _N_CHUNKS, body, 0)


def kernel(input_ids, table, gamma, beta):
    normed = _normalize_table(table, gamma, beta)
    flat_ids = input_ids.reshape(-1).astype(jnp.int32)
    out = _gather_kernel(normed, flat_ids)
    return out.reshape(input_ids.shape + (HIDDEN,))
